# baseline (speedup 1.0000x reference)
import functools

import jax
import jax.numpy as jnp
from jax import lax
from jax.experimental import pallas as pl
from jax.experimental.pallas import tpu as pltpu
from jax.experimental.pallas import tpu_sc as plsc

N_CHUNKS = 4
VB = 4096


def _sc_pool_chunk(x, emb_table, chunk, n_chunks):
    B, H = x.shape
    V, D = emb_table.shape
    CB = B // n_chunks
    info = plsc.get_sparse_core_info()
    NC, NS = info.num_cores, info.num_subcores
    NW = NC * NS
    b_per_w = CB // NW
    n_dreg = D // 16
    H0 = min(128, H)
    H1 = H - H0

    mesh = plsc.VectorSubcoreMesh(core_axis_name="c", subcore_axis_name="s")

    @functools.partial(
        pl.kernel,
        mesh=mesh,
        out_type=jax.ShapeDtypeStruct((CB, D), jnp.float32),
        scratch_types=[
            pltpu.VMEM((b_per_w, H), jnp.int32),
            pltpu.VMEM((2, H, D), jnp.float32),
            pltpu.VMEM((b_per_w, D), jnp.float32),
            pltpu.SemaphoreType.DMA((2,)),
        ],
        compiler_params=pltpu.CompilerParams(use_tc_tiling_on_sc=False),
    )
    def k(x_hbm, emb_hbm, out_hbm, idx_v, rows_v, out_v, sems):
        wid = lax.axis_index("s") * NC + lax.axis_index("c")
        base = wid * b_per_w
        scale = jnp.float32(1.0 / H)

        pltpu.sync_copy(x_hbm.at[pl.ds(chunk * CB + base, b_per_w)], idx_v)

        def gather(r, buf):
            return (
                pltpu.async_copy(
                    emb_hbm.at[idx_v.at[r, pl.ds(0, H0)]],
                    rows_v.at[buf, pl.ds(0, H0)],
                    sems.at[buf],
                ),
                pltpu.async_copy(
                    emb_hbm.at[idx_v.at[r, pl.ds(H0, H1)]],
                    rows_v.at[buf, pl.ds(H0, H1)],
                    sems.at[buf],
                ),
            )

        pending = {0: gather(0, 0)}
        if b_per_w > 1:
            pending[1] = gather(1, 1)
        for r in range(b_per_w):
            buf = r & 1
            for cp in pending.pop(r):
                cp.wait()

            def inner(j, accs):
                a = tuple(
                    accs[d] + rows_v[buf, 2 * j, pl.ds(d * 16, 16)]
                    for d in range(n_dreg)
                )
                return tuple(
                    a[d] + rows_v[buf, 2 * j + 1, pl.ds(d * 16, 16)]
                    for d in range(n_dreg)
                )

            accs = lax.fori_loop(
                0, H // 2, inner,
                tuple(jnp.zeros((16,), jnp.float32) for _ in range(n_dreg)),
            )
            if r + 2 < b_per_w:
                pending[r + 2] = gather(r + 2, buf)
            for d in range(n_dreg):
                out_v[r, pl.ds(d * 16, 16)] = accs[d] * scale

        pltpu.sync_copy(out_v, out_hbm.at[pl.ds(base, b_per_w)])

    return k(x, emb_table)


def _tc_matmul_chunk(m, W, b2, out_prev, chunk, B):
    CB, D = m.shape
    V = W.shape[0]
    nv = pl.cdiv(V, VB)

    def mm(*refs):
        m_ref, w_ref, b_ref, o_ref = refs[-4:]
        o_ref[...] = (
            lax.dot_general(
                m_ref[...], w_ref[...],
                (((1,), (1,)), ((), ())),
                preferred_element_type=jnp.float32,
            )
            + b_ref[...]
        )

    in_specs = [
        pl.BlockSpec((CB, D), lambda v: (0, 0)),
        pl.BlockSpec((VB, D), lambda v: (v, 0)),
        pl.BlockSpec((1, VB), lambda v: (0, v)),
    ]
    args = [m, W, b2]
    io_aliases = {}
    if out_prev is not None:
        in_specs = [pl.BlockSpec(memory_space=pl.ANY)] + in_specs
        args = [out_prev] + args
        io_aliases = {0: 0}

    return pl.pallas_call(
        mm,
        grid=(nv,),
        in_specs=in_specs,
        out_specs=pl.BlockSpec((CB, VB), lambda v: (chunk, v)),
        out_shape=jax.ShapeDtypeStruct((B, V), jnp.float32),
        input_output_aliases=io_aliases,
    )(*args)


def _tc_matmul_T(WT, m, b2):
    D, V = WT.shape
    B = m.shape[0]
    VB = 4096
    nv = pl.cdiv(V, VB)

    def mm(wt_ref, m_ref, b_ref, o_ref):
        o_ref[...] = (
            lax.dot_general(
                wt_ref[...], m_ref[...],
                (((0,), (1,)), ((), ())),
                preferred_element_type=jnp.float32,
            )
            + b_ref[...].T
        )

    return pl.pallas_call(
        mm,
        grid=(nv,),
        in_specs=[
            pl.BlockSpec((D, VB), lambda v: (0, v)),
            pl.BlockSpec((B, D), lambda v: (0, 0)),
            pl.BlockSpec((1, VB), lambda v: (0, v)),
        ],
        out_specs=pl.BlockSpec((VB, B), lambda v: (v, 0)),
        out_shape=jax.ShapeDtypeStruct((V, B), jnp.float32),
    )(WT, m, b2)


def kernel(x, emb_table, W, b):
    xi = jnp.maximum(x.astype(jnp.int32), 0)
    V = W.shape[0]
    b2 = b.reshape(1, V)
    m = _sc_pool_chunk(xi, emb_table, 0, 1)
    outT = _tc_matmul_T(W.T, m, b2)
    return outT.T


# speedup vs baseline: 1.0001x; 1.0001x over previous
import functools

import jax
import jax.numpy as jnp
from jax import lax
from jax.experimental import pallas as pl
from jax.experimental.pallas import tpu as pltpu
from jax.experimental.pallas import tpu_sc as plsc

N_CHUNKS = 4
VB = 4096


def _sc_pool_chunk(x, emb_table, chunk, n_chunks):
    B, H = x.shape
    V, D = emb_table.shape
    CB = B // n_chunks
    info = plsc.get_sparse_core_info()
    NC, NS = info.num_cores, info.num_subcores
    NW = NC * NS
    b_per_w = CB // NW
    n_dreg = D // 16
    H0 = min(128, H)
    H1 = H - H0

    mesh = plsc.VectorSubcoreMesh(core_axis_name="c", subcore_axis_name="s")

    @functools.partial(
        pl.kernel,
        mesh=mesh,
        out_type=jax.ShapeDtypeStruct((CB, D), jnp.float32),
        scratch_types=[
            pltpu.VMEM((b_per_w, H), jnp.int32),
            pltpu.VMEM((2, H, D), jnp.float32),
            pltpu.VMEM((b_per_w, D), jnp.float32),
            pltpu.SemaphoreType.DMA((2,)),
        ],
        compiler_params=pltpu.CompilerParams(use_tc_tiling_on_sc=False),
    )
    def k(x_hbm, emb_hbm, out_hbm, idx_v, rows_v, out_v, sems):
        wid = lax.axis_index("s") * NC + lax.axis_index("c")
        base = wid * b_per_w
        scale = jnp.float32(1.0 / H)

        pltpu.sync_copy(x_hbm.at[pl.ds(chunk * CB + base, b_per_w)], idx_v)

        def gather(r, buf):
            return (
                pltpu.async_copy(
                    emb_hbm.at[idx_v.at[r, pl.ds(0, H0)]],
                    rows_v.at[buf, pl.ds(0, H0)],
                    sems.at[buf],
                ),
                pltpu.async_copy(
                    emb_hbm.at[idx_v.at[r, pl.ds(H0, H1)]],
                    rows_v.at[buf, pl.ds(H0, H1)],
                    sems.at[buf],
                ),
            )

        pending = {0: gather(0, 0)}
        if b_per_w > 1:
            pending[1] = gather(1, 1)
        for r in range(b_per_w):
            buf = r & 1
            for cp in pending.pop(r):
                cp.wait()

            def inner(j, accs):
                a = tuple(
                    accs[d] + rows_v[buf, 2 * j, pl.ds(d * 16, 16)]
                    for d in range(n_dreg)
                )
                return tuple(
                    a[d] + rows_v[buf, 2 * j + 1, pl.ds(d * 16, 16)]
                    for d in range(n_dreg)
                )

            accs = lax.fori_loop(
                0, H // 2, inner,
                tuple(jnp.zeros((16,), jnp.float32) for _ in range(n_dreg)),
            )
            if r + 2 < b_per_w:
                pending[r + 2] = gather(r + 2, buf)
            for d in range(n_dreg):
                out_v[r, pl.ds(d * 16, 16)] = accs[d] * scale

        pltpu.sync_copy(out_v, out_hbm.at[pl.ds(base, b_per_w)])

    return k(x, emb_table)


def _tc_matmul_chunk(m, W, b2, out_prev, chunk, B):
    CB, D = m.shape
    V = W.shape[0]
    nv = pl.cdiv(V, VB)

    def mm(*refs):
        m_ref, w_ref, b_ref, o_ref = refs[-4:]
        o_ref[...] = (
            lax.dot_general(
                m_ref[...], w_ref[...],
                (((1,), (1,)), ((), ())),
                preferred_element_type=jnp.float32,
            )
            + b_ref[...]
        )

    in_specs = [
        pl.BlockSpec((CB, D), lambda v: (0, 0)),
        pl.BlockSpec((VB, D), lambda v: (v, 0)),
        pl.BlockSpec((1, VB), lambda v: (0, v)),
    ]
    args = [m, W, b2]
    io_aliases = {}
    if out_prev is not None:
        in_specs = [pl.BlockSpec(memory_space=pl.ANY)] + in_specs
        args = [out_prev] + args
        io_aliases = {0: 0}

    return pl.pallas_call(
        mm,
        grid=(nv,),
        in_specs=in_specs,
        out_specs=pl.BlockSpec((CB, VB), lambda v: (chunk, v)),
        out_shape=jax.ShapeDtypeStruct((B, V), jnp.float32),
        input_output_aliases=io_aliases,
    )(*args)


def _tc_matmul_T(WT, m, b2):
    D, V = WT.shape
    B = m.shape[0]
    VB = 4096
    nv = pl.cdiv(V, VB)

    def mm(wt_ref, m_ref, b_ref, o_ref):
        o_ref[...] = (
            lax.dot_general(
                wt_ref[...], m_ref[...],
                (((0,), (1,)), ((), ())),
                preferred_element_type=jnp.float32,
            )
            + b_ref[...].T
        )

    return pl.pallas_call(
        mm,
        grid=(nv,),
        in_specs=[
            pl.BlockSpec((D, VB), lambda v: (0, v)),
            pl.BlockSpec((B, D), lambda v: (0, 0)),
            pl.BlockSpec((1, VB), lambda v: (0, v)),
        ],
        out_specs=pl.BlockSpec((VB, B), lambda v: (v, 0)),
        out_shape=jax.ShapeDtypeStruct((V, B), jnp.float32),
    )(WT, m, b2)


def kernel(x, emb_table, W, b):
    xi = jnp.maximum(x.astype(jnp.int32), 0)
    emb_table = emb_table + jnp.float32(0.0)
    V = W.shape[0]
    b2 = b.reshape(1, V)
    m = _sc_pool_chunk(xi, emb_table, 0, 1)
    outT = _tc_matmul_T(W.T, m, b2)
    return outT.T


# speedup vs baseline: 1.0375x; 1.0373x over previous
import functools

import jax
import jax.numpy as jnp
from jax import lax
from jax.experimental import pallas as pl
from jax.experimental.pallas import tpu as pltpu
from jax.experimental.pallas import tpu_sc as plsc

N_CHUNKS = 4
VB = 4096


def _sc_pool_chunk(x, emb_table, chunk, n_chunks):
    B, H = x.shape
    V, D = emb_table.shape
    CB = B // n_chunks
    info = plsc.get_sparse_core_info()
    NC, NS = info.num_cores, info.num_subcores
    NW = NC * NS
    b_per_w = CB // NW
    n_dreg = D // 16
    H0 = min(128, H)
    H1 = H - H0

    mesh = plsc.VectorSubcoreMesh(core_axis_name="c", subcore_axis_name="s")

    @functools.partial(
        pl.kernel,
        mesh=mesh,
        out_type=jax.ShapeDtypeStruct((CB, D), jnp.float32),
        scratch_types=[
            pltpu.VMEM((b_per_w, H), jnp.int32),
            pltpu.VMEM((4, H, D), jnp.float32),
            pltpu.VMEM((b_per_w, D), jnp.float32),
            pltpu.SemaphoreType.DMA((4,)),
        ],
        compiler_params=pltpu.CompilerParams(use_tc_tiling_on_sc=False),
    )
    def k(x_hbm, emb_hbm, out_hbm, idx_v, rows_v, out_v, sems):
        wid = lax.axis_index("s") * NC + lax.axis_index("c")
        base = wid * b_per_w
        scale = jnp.float32(1.0 / H)

        pltpu.sync_copy(x_hbm.at[pl.ds(chunk * CB + base, b_per_w)], idx_v)

        def gather(r, buf):
            return (
                pltpu.async_copy(
                    emb_hbm.at[idx_v.at[r, pl.ds(0, H0)]],
                    rows_v.at[buf, pl.ds(0, H0)],
                    sems.at[buf],
                ),
                pltpu.async_copy(
                    emb_hbm.at[idx_v.at[r, pl.ds(H0, H1)]],
                    rows_v.at[buf, pl.ds(H0, H1)],
                    sems.at[buf],
                ),
            )

        NBUF = 4
        pending = {}
        for r0 in range(min(NBUF, b_per_w)):
            pending[r0] = gather(r0, r0)
        for r in range(b_per_w):
            buf = r % NBUF
            for cp in pending.pop(r):
                cp.wait()

            def inner(j, accs):
                for u in range(4):
                    accs = tuple(
                        accs[d] + rows_v[buf, 4 * j + u, pl.ds(d * 16, 16)]
                        for d in range(n_dreg)
                    )
                return accs

            accs = lax.fori_loop(
                0, H // 4, inner,
                tuple(jnp.zeros((16,), jnp.float32) for _ in range(n_dreg)),
            )
            if r + NBUF < b_per_w:
                pending[r + NBUF] = gather(r + NBUF, buf)
            for d in range(n_dreg):
                out_v[r, pl.ds(d * 16, 16)] = accs[d] * scale

        pltpu.sync_copy(out_v, out_hbm.at[pl.ds(base, b_per_w)])

    return k(x, emb_table)


def _tc_matmul_chunk(m, W, b2, out_prev, chunk, B):
    CB, D = m.shape
    V = W.shape[0]
    nv = pl.cdiv(V, VB)

    def mm(*refs):
        m_ref, w_ref, b_ref, o_ref = refs[-4:]
        o_ref[...] = (
            lax.dot_general(
                m_ref[...], w_ref[...],
                (((1,), (1,)), ((), ())),
                preferred_element_type=jnp.float32,
            )
            + b_ref[...]
        )

    in_specs = [
        pl.BlockSpec((CB, D), lambda v: (0, 0)),
        pl.BlockSpec((VB, D), lambda v: (v, 0)),
        pl.BlockSpec((1, VB), lambda v: (0, v)),
    ]
    args = [m, W, b2]
    io_aliases = {}
    if out_prev is not None:
        in_specs = [pl.BlockSpec(memory_space=pl.ANY)] + in_specs
        args = [out_prev] + args
        io_aliases = {0: 0}

    return pl.pallas_call(
        mm,
        grid=(nv,),
        in_specs=in_specs,
        out_specs=pl.BlockSpec((CB, VB), lambda v: (chunk, v)),
        out_shape=jax.ShapeDtypeStruct((B, V), jnp.float32),
        input_output_aliases=io_aliases,
    )(*args)


def _tc_matmul_T(WT, m, b2):
    D, V = WT.shape
    B = m.shape[0]
    VB = 4096
    nv = pl.cdiv(V, VB)

    def mm(wt_ref, m_ref, b_ref, o_ref):
        o_ref[...] = (
            lax.dot_general(
                wt_ref[...], m_ref[...],
                (((0,), (1,)), ((), ())),
                preferred_element_type=jnp.float32,
            )
            + b_ref[...].T
        )

    return pl.pallas_call(
        mm,
        grid=(nv,),
        in_specs=[
            pl.BlockSpec((D, VB), lambda v: (0, v)),
            pl.BlockSpec((B, D), lambda v: (0, 0)),
            pl.BlockSpec((1, VB), lambda v: (0, v)),
        ],
        out_specs=pl.BlockSpec((VB, B), lambda v: (v, 0)),
        out_shape=jax.ShapeDtypeStruct((V, B), jnp.float32),
    )(WT, m, b2)


def kernel(x, emb_table, W, b):
    xi = jnp.maximum(x.astype(jnp.int32), 0)
    V = W.shape[0]
    b2 = b.reshape(1, V)
    m = _sc_pool_chunk(xi, emb_table, 0, 1)
    outT = _tc_matmul_T(W.T, m, b2)
    return outT.T
